# Initial kernel scaffold; baseline (speedup 1.0000x reference)
import jax
import jax.numpy as jnp
from jax.experimental import pallas as pl


def kernel(x, term_walk_index, W_T, b_T, W_M, b_M, W_B, b_B, W_1, b_1, W_t, b_t):
    raise NotImplementedError("write your pallas kernel here")


# speedup vs baseline: 2.9730x; 2.9730x over previous
import functools

import jax
import jax.numpy as jnp
from jax import lax
from jax.experimental import pallas as pl
from jax.experimental.pallas import tpu as pltpu
from jax.experimental.pallas import tpu_sc as plsc

N = 10000
E = 320000
D = 128
K = 2
EPS = 1e-5

C = 3 * D + 1
CP = 512
TN = 3 * N
NP = 10240

NC, NS, L = 2, 16, 16
NW = NC * NS
NG = 25

EPW = E // NW
B1 = 16
NB1 = EPW // B1

EPS2 = E // NS
B2 = 80
NB2 = EPS2 // B2
EPS2B = E // NW
NB2B = EPS2B // B2

RB = 1000
NBA = N // RB
ER = E // L
ERB = 2000
NBB = ER // ERB

RPS = NP // NS
ZR = RPS // 5

_MESH = plsc.VectorSubcoreMesh(
    core_axis_name="c", subcore_axis_name="s", num_cores=NC, num_subcores=NS
)


def _wid():
    return lax.axis_index("s") * NC + lax.axis_index("c")


@functools.partial(
    pl.kernel,
    mesh=_MESH,
    out_type=jax.ShapeDtypeStruct((NW * TN,), jnp.float32),
    scratch_types=[
        pltpu.VMEM((EPW,), jnp.int32),
        pltpu.VMEM((EPW,), jnp.int32),
        pltpu.VMEM((EPW,), jnp.int32),
        pltpu.VMEM((TN,), jnp.float32),
    ],
    compiler_params=pltpu.CompilerParams(needs_layout_passes=False),
)
def _sc_counts(jflat_hbm, cnt_hbm, j0, j1, j2, hist):
    wid = _wid()
    base = wid * EPW
    js = (j0, j1, j2)

    def zero_body(k, _):
        hist[pl.ds(k * L, L)] = jnp.zeros((L,), jnp.float32)
        return 0

    lax.fori_loop(0, TN // L, zero_body, 0)

    for p in range(3):
        pltpu.sync_copy(jflat_hbm.at[pl.ds(p * E + base, EPW)], js[p])

    ones = jnp.ones((L,), jnp.float32)
    for p in range(3):
        def hbody(k, _, p=p):
            iv = js[p][pl.ds(k * L, L)]
            plsc.addupdate_scatter(hist, [iv], ones)
            return 0

        lax.fori_loop(0, EPW // L, hbody, 0)

    pltpu.sync_copy(hist, cnt_hbm.at[pl.ds(wid * TN, TN)])


def _tc_cnt_body(c_ref, o_ref):
    o_ref[...] = jnp.sum(c_ref[...], axis=0)


def _tc_cnt(cnt2):
    return pl.pallas_call(
        _tc_cnt_body,
        in_specs=[pl.BlockSpec((NW, TN), lambda: (0, 0))],
        out_specs=pl.BlockSpec((TN,), lambda: (0,)),
        out_shape=jax.ShapeDtypeStruct((TN,), jnp.float32),
    )(cnt2)


def _tc_proj_body(x_ref, w_ref, c_ref, p_ref, pm_ref):
    P = jnp.dot(x_ref[...], w_ref[0], preferred_element_type=jnp.float32)
    p_ref[0] = P
    cvec = c_ref[...]
    pm_ref[0, 0, 0] = jnp.sum(P * cvec, axis=0)


def _tc_proj(x, wcat, cnt3d):
    return pl.pallas_call(
        _tc_proj_body,
        grid=(3, NBA),
        in_specs=[
            pl.BlockSpec((RB, D), lambda p, b: (b, 0)),
            pl.BlockSpec((1, D, CP), lambda p, b: (p, 0, 0)),
            pl.BlockSpec((RB, 1), lambda p, b: (p * NBA + b, 0)),
        ],
        out_specs=[
            pl.BlockSpec((1, RB, CP), lambda p, b: (p, b, 0)),
            pl.BlockSpec((1, 1, 1, CP), lambda p, b: (p, b, 0, 0)),
        ],
        out_shape=[
            jax.ShapeDtypeStruct((3, N, CP), jnp.float32),
            jax.ShapeDtypeStruct((3, NBA, 1, CP), jnp.float32),
        ],
    )(x, wcat, cnt3d)


@functools.partial(
    pl.kernel,
    mesh=_MESH,
    out_type=(
        jax.ShapeDtypeStruct((E, D), jnp.float32),
        jax.ShapeDtypeStruct((E, D), jnp.float32),
        jax.ShapeDtypeStruct((E, D), jnp.float32),
        jax.ShapeDtypeStruct((E, L), jnp.float32),
        jax.ShapeDtypeStruct((NW * CP,), jnp.float32),
        jax.ShapeDtypeStruct((NW * L,), jnp.float32),
    ),
    scratch_types=[
        pltpu.VMEM((EPW,), jnp.int32),
        pltpu.VMEM((EPW,), jnp.int32),
        pltpu.VMEM((EPW,), jnp.int32),
        pltpu.VMEM((B1, CP), jnp.float32),
        pltpu.VMEM((B1, CP), jnp.float32),
        pltpu.VMEM((B1, CP), jnp.float32),
        pltpu.VMEM((B1, CP), jnp.float32),
        pltpu.VMEM((B1, CP), jnp.float32),
        pltpu.VMEM((B1, CP), jnp.float32),
        pltpu.VMEM((B1, 3 * D), jnp.float32),
        pltpu.VMEM((B1, 3 * D), jnp.float32),
        pltpu.VMEM((B1, L), jnp.float32),
        pltpu.VMEM((B1, L), jnp.float32),
        pltpu.VMEM((CP,), jnp.float32),
        pltpu.VMEM((L,), jnp.float32),
        pltpu.SemaphoreType.DMA,
        pltpu.SemaphoreType.DMA,
        pltpu.SemaphoreType.DMA,
        pltpu.SemaphoreType.DMA,
    ],
    compiler_params=pltpu.CompilerParams(needs_layout_passes=False),
)
def _sc_pass1(ptab_hbm, jflat_hbm, zt_hbm, zm_hbm, zb_hbm, scw_hbm, sq_hbm,
              mx_hbm, j0, j1, j2, ga0, ga1, ga2, gb0, gb1, gb2, zsa, zsb,
              ssa, ssb, qbuf, mxb, semga, semgb, semsa, semsb):
    wid = _wid()
    base_w = wid * EPW
    js = (j0, j1, j2)
    for p in range(3):
        pltpu.sync_copy(jflat_hbm.at[pl.ds(p * E + base_w, EPW)], js[p])

    zero = jnp.zeros((L,), jnp.float32)
    ninf = jnp.full((L,), -jnp.inf, jnp.float32)

    def fire_gathers(nb, bufs, sem):
        eb = nb * B1
        pltpu.async_copy(ptab_hbm.at[j0.at[pl.ds(eb, B1)]], bufs[0], sem)
        pltpu.async_copy(ptab_hbm.at[j1.at[pl.ds(eb, B1)]], bufs[1], sem)
        pltpu.async_copy(ptab_hbm.at[j2.at[pl.ds(eb, B1)]], bufs[2], sem)

    def drain_gathers(bufs, sem):
        for b in bufs:
            pltpu.make_async_copy(ptab_hbm.at[pl.ds(0, B1)], b, sem).wait()

    def fire_stores(nb, zs, ss, sem):
        rows = pl.ds(base_w + nb * B1, B1)
        pltpu.async_copy(zs.at[:, pl.ds(0, D)], zt_hbm.at[rows], sem)
        pltpu.async_copy(zs.at[:, pl.ds(D, D)], zm_hbm.at[rows], sem)
        pltpu.async_copy(zs.at[:, pl.ds(2 * D, D)], zb_hbm.at[rows], sem)
        pltpu.async_copy(ss, scw_hbm.at[rows], sem)

    def drain_stores(zs, ss, sem):
        rows = pl.ds(0, B1)
        pltpu.make_async_copy(zs.at[:, pl.ds(0, D)], zt_hbm.at[rows],
                              sem).wait()
        pltpu.make_async_copy(zs.at[:, pl.ds(D, D)], zm_hbm.at[rows],
                              sem).wait()
        pltpu.make_async_copy(zs.at[:, pl.ds(2 * D, D)], zb_hbm.at[rows],
                              sem).wait()
        pltpu.make_async_copy(ss, scw_hbm.at[rows], sem).wait()

    def compute(bufs, zs, ss, carry):
        qs, mx = carry

        @plsc.parallel_loop(0, B1, unroll=1, carry=(qs, mx))
        def edge_res(e, ecarry):
            eqs, emx = ecarry
            nqs = []
            for g in range(NG):
                sl = pl.ds(g * L, L)
                v = bufs[0][e, sl] + bufs[1][e, sl] + bufs[2][e, sl]
                nqs.append(eqs[g] + v * v)
                if g < NG - 1:
                    zs[e, sl] = v
                else:
                    ss[e, pl.ds(0, L)] = v
                    emx = jnp.maximum(emx, v)
            return tuple(nqs), emx

        return edge_res

    ga = (ga0, ga1, ga2)
    gb = (gb0, gb1, gb2)
    fire_gathers(0, ga, semga)

    def pair_body(k2, carry):
        b0 = k2 * 2
        fire_gathers(b0 + 1, gb, semgb)
        drain_gathers(ga, semga)

        @pl.when(k2 > 0)
        def _():
            drain_stores(zsa, ssa, semsa)

        carry = compute(ga, zsa, ssa, carry)
        fire_stores(b0, zsa, ssa, semsa)
        fire_gathers(b0 + 2, ga, semga)

        drain_gathers(gb, semgb)

        @pl.when(k2 > 0)
        def _():
            drain_stores(zsb, ssb, semsb)

        carry = compute(gb, zsb, ssb, carry)
        fire_stores(b0 + 1, zsb, ssb, semsb)
        return carry

    qs0 = tuple(zero for _ in range(NG))
    qs, mx = lax.fori_loop(0, NB1 // 2, pair_body, (qs0, ninf))

    drain_gathers(ga, semga)
    drain_stores(zsa, ssa, semsa)
    qs, mx = compute(ga, zsa, ssa, (qs, mx))
    fire_stores(NB1 - 1, zsa, ssa, semsa)
    drain_stores(zsa, ssa, semsa)
    drain_stores(zsb, ssb, semsb)

    for g in range(NG):
        qbuf[pl.ds(g * L, L)] = qs[g]
    for g in range(NG, CP // L):
        qbuf[pl.ds(g * L, L)] = zero
    mxb[pl.ds(0, L)] = mx
    pltpu.sync_copy(qbuf, sq_hbm.at[pl.ds(wid * CP, CP)])
    pltpu.sync_copy(mxb, mx_hbm.at[pl.ds(wid * L, L)])


def _tc_b1_body(sv_ref, sq_ref, mx_ref, pm_ref, exw_ref, ps_ref, par_ref):
    pms = jnp.sum(pm_ref[...], axis=(0, 1, 2))
    mean = pms * (1.0 / E)
    sq = jnp.sum(sq_ref[...], axis=0)
    var = sq * (1.0 / E) - mean * mean
    rinv = lax.rsqrt(var + EPS)
    m1 = mean[3 * D]
    r1 = rinv[3 * D]
    smax = jnp.max(mx_ref[..., 0])
    M = jnp.maximum(0.0, (smax - m1) * r1)

    sv = sv_ref[...]
    ex = jnp.exp(jnp.maximum(0.0, (sv - m1) * r1) - M)
    cols = lax.broadcasted_iota(jnp.int32, ex.shape, 1)
    exm = jnp.where(cols % L == 0, ex, 0.0)
    exw_ref[...] = exm
    ps_ref[0, 0] = jnp.full((8,), jnp.sum(exm), jnp.float32)
    par_ref[0, :] = mean
    par_ref[1, :] = rinv
    par_ref[2:8, :] = jnp.zeros((6, CP), jnp.float32)


def _tc_b1(score_v, sq_part, mx_part, pmean_part):
    return pl.pallas_call(
        _tc_b1_body,
        grid=(NBB,),
        in_specs=[
            pl.BlockSpec((ERB, 256), lambda b: (b, 0)),
            pl.BlockSpec((NW, CP), lambda b: (0, 0)),
            pl.BlockSpec((NW, L), lambda b: (0, 0)),
            pl.BlockSpec((3, NBA, 1, CP), lambda b: (0, 0, 0, 0)),
        ],
        out_specs=[
            pl.BlockSpec((ERB, 256), lambda b: (b, 0)),
            pl.BlockSpec((1, 1, 8), lambda b: (b, 0, 0)),
            pl.BlockSpec((8, CP), lambda b: (0, 0)),
        ],
        out_shape=[
            jax.ShapeDtypeStruct((ER, 256), jnp.float32),
            jax.ShapeDtypeStruct((NBB, 1, 8), jnp.float32),
            jax.ShapeDtypeStruct((8, CP), jnp.float32),
        ],
    )(score_v, sq_part, mx_part, pmean_part)


def _tc_b2_body(exw_ref, ps_ref, attw_ref):
    denom = jnp.sum(ps_ref[..., 0])
    ii = lax.broadcasted_iota(jnp.int32, (256, 256), 0)
    jj = lax.broadcasted_iota(jnp.int32, (256, 256), 1)
    spl = jnp.where((ii % L == 0) & (ii // L == jj // L), 1.0, 0.0)
    attw_ref[...] = jnp.dot(
        exw_ref[...], spl, preferred_element_type=jnp.float32
    ) * (1.0 / denom)


def _tc_b2(exw, psum):
    return pl.pallas_call(
        _tc_b2_body,
        grid=(NBB,),
        in_specs=[
            pl.BlockSpec((ERB, 256), lambda b: (b, 0)),
            pl.BlockSpec((NBB, 1, 8), lambda b: (0, 0, 0)),
        ],
        out_specs=pl.BlockSpec((ERB, 256), lambda b: (b, 0)),
        out_shape=jax.ShapeDtypeStruct((ER, 256), jnp.float32),
    )(exw, psum)


@functools.partial(
    pl.kernel,
    mesh=_MESH,
    out_type=(
        jax.ShapeDtypeStruct((NP, D), jnp.float32),
        jax.ShapeDtypeStruct((NP, D), jnp.float32),
        jax.ShapeDtypeStruct((NC, NP, D), jnp.float32),
    ),
    scratch_types=[
        pltpu.VMEM((B2, D), jnp.float32),
        pltpu.VMEM((B2, L), jnp.float32),
        pltpu.VMEM((B2,), jnp.int32),
        pltpu.VMEM((B2, D), jnp.float32),
        pltpu.VMEM((B2, L), jnp.float32),
        pltpu.VMEM((B2,), jnp.int32),
        pltpu.VMEM((8, CP), jnp.float32),
        pltpu.VMEM_SHARED((NP, D), jnp.float32),
        pltpu.SemaphoreType.DMA,
        pltpu.SemaphoreType.DMA,
        pltpu.SemaphoreType.DMA,
        pltpu.SemaphoreType.DMA,
    ],
    compiler_params=pltpu.CompilerParams(needs_layout_passes=False),
)
def _sc_pass2(zt_hbm, zm_hbm, zb_hbm, attw_hbm, iflat_hbm, par_hbm,
              zeros_hbm, at_hbm, am_hbm, ab_hbm, zba, aba, ixa, zbb, abb,
              ixb, pbuf, acc_sh, semla, semlb, semca, semcb):
    cid = lax.axis_index("c")
    sid = lax.axis_index("s")
    rows_own = pl.ds(sid * RPS, RPS)

    pltpu.sync_copy(par_hbm, pbuf)

    def zero_own():
        pltpu.sync_copy(zeros_hbm, acc_sh.at[rows_own])

    def scatter_phase(z_hbm, nblocks, base_edges, idx_off, off):
        means = [pbuf[0, pl.ds(off + g * L, L)] for g in range(D // L)]
        rinvs = [pbuf[1, pl.ds(off + g * L, L)] for g in range(D // L)]

        def fire_load(nb, zb, ab, ix, sem):
            rows = pl.ds(base_edges + nb * B2, B2)
            pltpu.async_copy(z_hbm.at[rows], zb, sem)
            pltpu.async_copy(attw_hbm.at[rows], ab, sem)
            pltpu.async_copy(
                iflat_hbm.at[pl.ds(idx_off + base_edges + nb * B2, B2)],
                ix, sem)

        def drain_load(zb, ab, ix, sem):
            rows = pl.ds(0, B2)
            pltpu.make_async_copy(z_hbm.at[rows], zb, sem).wait()
            pltpu.make_async_copy(attw_hbm.at[rows], ab, sem).wait()
            pltpu.make_async_copy(iflat_hbm.at[rows], ix, sem).wait()

        def drain_scat(zb, ix, sem):
            del ix
            pltpu.make_async_copy(z_hbm.at[pl.ds(0, B2)], zb, sem).wait()

        def work(zb, ab, ix, sem):
            @plsc.parallel_loop(0, B2, unroll=2)
            def _(e):
                av = ab[e, pl.ds(0, L)]
                for g in range(D // L):
                    sl = pl.ds(g * L, L)
                    v = (zb[e, sl] - means[g]) * rinvs[g]
                    v = jnp.maximum(v, 0.0) * av
                    zb[e, sl] = v

            pltpu.async_copy(zb, acc_sh.at[ix], sem, add=True)

        npair = nblocks // 2
        fire_load(0, zba, aba, ixa, semla)

        def pair(k2, _):
            b0 = k2 * 2

            @pl.when(k2 > 0)
            def _():
                drain_scat(zbb, ixb, semcb)

            fire_load(b0 + 1, zbb, abb, ixb, semlb)
            drain_load(zba, aba, ixa, semla)
            work(zba, aba, ixa, semca)
            drain_load(zbb, abb, ixb, semlb)

            @pl.when(b0 + 2 < nblocks)
            def _():
                drain_scat(zba, ixa, semca)
                fire_load(b0 + 2, zba, aba, ixa, semla)

            work(zbb, abb, ixb, semcb)
            return 0

        lax.fori_loop(0, npair, pair, 0)
        if nblocks % 2 == 1:
            drain_load(zba, aba, ixa, semla)
            work(zba, aba, ixa, semca)
        drain_scat(zba, ixa, semca)
        drain_scat(zbb, ixb, semcb)

    zero_own()
    plsc.subcore_barrier()

    @pl.when(cid == 0)
    def _():
        scatter_phase(zt_hbm, NB2, sid * EPS2, 0, 0)

    @pl.when(cid == 1)
    def _():
        scatter_phase(zm_hbm, NB2, sid * EPS2, E, D)

    plsc.subcore_barrier()

    @pl.when(cid == 0)
    def _():
        pltpu.sync_copy(acc_sh.at[rows_own], at_hbm.at[rows_own])

    @pl.when(cid == 1)
    def _():
        pltpu.sync_copy(acc_sh.at[rows_own], am_hbm.at[rows_own])

    zero_own()
    plsc.subcore_barrier()
    scatter_phase(zb_hbm, NB2B, cid * (E // NC) + sid * EPS2B, 2 * E, 2 * D)
    plsc.subcore_barrier()
    pltpu.sync_copy(acc_sh.at[rows_own], ab_hbm.at[cid, rows_own])


def _tc_c1_body(at_ref, am_ref, ab_ref, c0_ref, c1_ref, c2_ref, wt_ref,
                y_ref, yp_ref):
    i0 = 1.0 / jnp.maximum(c0_ref[...], 1.0)
    i1 = 1.0 / jnp.maximum(c1_ref[...], 1.0)
    i2 = 1.0 / jnp.maximum(c2_ref[...], 1.0)
    m = (at_ref[...] * i0 + am_ref[...] * i1
         + (ab_ref[0] + ab_ref[1]) * i2)
    y = jnp.dot(m, wt_ref[...], preferred_element_type=jnp.float32)
    y_ref[...] = y
    yp_ref[0, 0] = jnp.sum(y, axis=0)
    yp_ref[0, 1] = jnp.sum(y * y, axis=0)


def _tc_c1(acc_t, acc_m, acc_b, c0, c1, c2, wt_t):
    return pl.pallas_call(
        _tc_c1_body,
        grid=(NBA,),
        in_specs=[
            pl.BlockSpec((RB, D), lambda b: (b, 0)),
            pl.BlockSpec((RB, D), lambda b: (b, 0)),
            pl.BlockSpec((NC, RB, D), lambda b: (0, b, 0)),
            pl.BlockSpec((RB, 1), lambda b: (b, 0)),
            pl.BlockSpec((RB, 1), lambda b: (b, 0)),
            pl.BlockSpec((RB, 1), lambda b: (b, 0)),
            pl.BlockSpec((D, D), lambda b: (0, 0)),
        ],
        out_specs=[
            pl.BlockSpec((RB, D), lambda b: (b, 0)),
            pl.BlockSpec((1, 2, D), lambda b: (b, 0, 0)),
        ],
        out_shape=[
            jax.ShapeDtypeStruct((N, D), jnp.float32),
            jax.ShapeDtypeStruct((NBA, 2, D), jnp.float32),
        ],
    )(acc_t, acc_m, acc_b, c0, c1, c2, wt_t)


def _tc_c2_body(y_ref, yp_ref, x_ref, o_ref):
    s = jnp.sum(yp_ref[:, 0, :], axis=0)
    q = jnp.sum(yp_ref[:, 1, :], axis=0)
    mean = s * (1.0 / N)
    var = q * (1.0 / N) - mean * mean
    rinv = lax.rsqrt(var + EPS)
    o_ref[...] = x_ref[...] + jnp.maximum(0.0, (y_ref[...] - mean) * rinv)


def _tc_c2(y, ypart, x):
    return pl.pallas_call(
        _tc_c2_body,
        grid=(NBA,),
        in_specs=[
            pl.BlockSpec((RB, D), lambda b: (b, 0)),
            pl.BlockSpec((NBA, 2, D), lambda b: (0, 0, 0)),
            pl.BlockSpec((RB, D), lambda b: (b, 0)),
        ],
        out_specs=pl.BlockSpec((RB, D), lambda b: (b, 0)),
        out_shape=jax.ShapeDtypeStruct((N, D), jnp.float32),
    )(y, ypart, x)


def kernel(x, term_walk_index, W_T, b_T, W_M, b_M, W_B, b_B, W_1, b_1,
           W_t, b_t):
    idx = term_walk_index.astype(jnp.int32)
    iflat = idx.reshape(3 * E)
    offs = (jnp.arange(3, dtype=jnp.int32) * N)[:, None]
    jflat = (idx + offs).reshape(3 * E)

    zeros_rps = jnp.zeros((RPS, D), jnp.float32)
    cnt2 = _sc_counts(jflat).reshape(NW, TN)
    cnts = _tc_cnt(cnt2)
    cnt3d = cnts[:, None]
    c_stack = cnts.reshape(3, N, 1)
    c0 = c_stack[0]
    c1 = c_stack[1]
    c2 = c_stack[2]

    for i in range(K):
        ws = jnp.concatenate([W_T[i], W_M[i], W_B[i], W_1[i]], axis=0)
        ws = jnp.concatenate(
            [ws, jnp.zeros((CP - C, 3 * D), jnp.float32)], axis=0
        )
        wcat = jnp.stack(
            [ws[:, p * D:(p + 1) * D].T for p in range(3)], axis=0
        )

        ptab3, pmean_part = _tc_proj(x, wcat, cnt3d)
        ptab = ptab3.reshape(TN, CP)

        zt, zm, zb, scw, sq_part, mx_part = _sc_pass1(ptab, jflat)

        score_v = scw.reshape(ER, 256)
        exw, psum, params = _tc_b1(
            score_v, sq_part.reshape(NW, CP), mx_part.reshape(NW, L),
            pmean_part
        )
        attw = _tc_b2(exw, psum).reshape(E, L)

        acc_t, acc_m, acc_b = _sc_pass2(zt, zm, zb, attw, iflat, params,
                                        zeros_rps)

        y, ypart = _tc_c1(acc_t, acc_m, acc_b, c0, c1, c2, W_t[i].T)
        x = _tc_c2(y, ypart, x)

    return x
